# baseline (speedup 1.0000x reference)
import functools

import jax
import jax.numpy as jnp
from jax import lax
from jax.experimental import pallas as pl
from jax.experimental.pallas import tpu as pltpu
from jax.experimental.pallas import tpu_sc as plsc


def _make_lookup(vocab, embed, batch):
    info = plsc.get_sparse_core_info()
    num_cores, num_subcores = info.num_cores, info.num_subcores
    num_workers = num_cores * num_subcores
    assert batch % num_workers == 0
    n = batch // num_workers

    mesh = plsc.VectorSubcoreMesh(core_axis_name="c", subcore_axis_name="s")

    @functools.partial(
        pl.kernel,
        mesh=mesh,
        compiler_params=pltpu.CompilerParams(use_tc_tiling_on_sc=False),
        out_type=jax.ShapeDtypeStruct((batch, 2 * embed), jnp.float32),
        scratch_types=[
            pltpu.VMEM((n,), jnp.int32),
            pltpu.VMEM((n,), jnp.int32),
            pltpu.VMEM((n, embed), jnp.float32),
            pltpu.VMEM((n, embed), jnp.float32),
            pltpu.SemaphoreType.DMA,
            pltpu.SemaphoreType.DMA,
        ],
    )
    def lookup(user_hbm, item_hbm, table_hbm, out_hbm,
               idx_u, idx_i, rows_u, rows_i, sem_u, sem_i):
        wid = lax.axis_index("s") * num_cores + lax.axis_index("c")
        base = wid * n
        pltpu.sync_copy(user_hbm.at[pl.ds(base, n)], idx_u)
        pltpu.sync_copy(item_hbm.at[pl.ds(base, n)], idx_i)
        cp_u = pltpu.async_copy(table_hbm.at[idx_u], rows_u, sem_u)
        cp_i = pltpu.async_copy(table_hbm.at[idx_i], rows_i, sem_i)
        cp_u.wait()
        pltpu.sync_copy(rows_u, out_hbm.at[pl.ds(base, n), pl.ds(0, embed)])
        cp_i.wait()
        pltpu.sync_copy(rows_i, out_hbm.at[pl.ds(base, n), pl.ds(embed, embed)])

    return lookup


def kernel(user, item, hash_embeds_weight):
    vocab, embed = hash_embeds_weight.shape
    (batch,) = user.shape
    lookup = _make_lookup(vocab, embed, batch)
    return lookup(user, item, hash_embeds_weight)
